# baseline (speedup 1.0000x reference)
import jax
import jax.numpy as jnp
import numpy as np
from jax import lax
from jax.experimental import pallas as pl
from jax.experimental.pallas import tpu as pltpu
from jax.experimental.pallas import tpu_sc as plsc

_VOCAB = 1000000
_D = 128
_SEQ = 8192

_NC = 2
_NS = 16
_L = 16
_NW = _NC * _NS
_B_PER_W = _SEQ // _NW
_K = 8
_R = _B_PER_W // _K
_NCH = _D // _L


def _coef_table():
    col = np.arange(_D)
    w = np.power(10000.0, -2.0 * (col // 2) / _D)
    even = (col % 2) == 0

    steps = np.concatenate([np.cos(w), np.sin(w)])

    anchors = np.empty((_NW, _K, 2, _D))
    for wid in range(_NW):
        for k in range(_K):
            p0 = wid * _B_PER_W + k * _R
            s, c = np.sin(p0 * w), np.cos(p0 * w)
            anchors[wid, k, 0] = np.where(even, s, c)
            anchors[wid, k, 1] = np.where(even, c, -s)
    flat = np.concatenate([steps, anchors.reshape(-1)])
    return flat.astype(np.float32)


_COEF = jnp.asarray(_coef_table())
_ANC_PER_W = _K * 2 * _D


def _body(tok_hbm, idx_hbm, coef_hbm, out_hbm,
          idx_v, rows_v, step_v, anc_v, *sems):
    gsems, wsem = sems[:_K], sems[_K]
    wid = lax.axis_index("s") * _NC + lax.axis_index("c")
    base = wid * _B_PER_W

    pltpu.sync_copy(idx_hbm.at[pl.ds(base, _B_PER_W)], idx_v)
    pltpu.sync_copy(coef_hbm.at[pl.ds(0, 2 * _D)], step_v)
    pltpu.sync_copy(
        coef_hbm.at[pl.ds(2 * _D + wid * _ANC_PER_W, _ANC_PER_W)], anc_v)
    gathers = [
        pltpu.async_copy(tok_hbm.at[idx_v.at[pl.ds(k * _R, _R)]],
                         rows_v.at[k], gsems[k])
        for k in range(_K)
    ]

    cw = [step_v[pl.ds(c * _L, _L)] for c in range(_NCH)]
    sw = [step_v[pl.ds(_D + c * _L, _L)] for c in range(_NCH)]

    def add_chunk(k):
        def add_row(r, st):
            v, u = st
            nv, nu = [], []
            for c in range(_NCH):
                plsc.addupdate(rows_v.at[k, r, pl.ds(c * _L, _L)], v[c])
                nv.append(v[c] * cw[c] + u[c] * sw[c])
                nu.append(u[c] * cw[c] - v[c] * sw[c])
            return tuple(nv), tuple(nu)

        v0 = tuple(anc_v[pl.ds(k * 2 * _D + c * _L, _L)]
                   for c in range(_NCH))
        u0 = tuple(anc_v[pl.ds(k * 2 * _D + _D + c * _L, _L)]
                   for c in range(_NCH))
        lax.fori_loop(0, _R, add_row, (v0, u0), unroll=2)

    writes = []
    for k in range(_K):
        gathers[k].wait()
        add_chunk(k)
        writes.append(pltpu.async_copy(
            rows_v.at[k], out_hbm.at[pl.ds(base + k * _R, _R)], wsem))
    for wdma in writes:
        wdma.wait()


def _embed(x, tok_table, coef):
    mesh = plsc.VectorSubcoreMesh(
        core_axis_name="c", subcore_axis_name="s",
        num_cores=_NC, num_subcores=_NS)
    return pl.kernel(
        _body,
        out_type=jax.ShapeDtypeStruct((_SEQ, _D), jnp.float32),
        mesh=mesh,
        scratch_types=(
            [pltpu.VMEM((_B_PER_W,), jnp.int32),
             pltpu.VMEM((_K, _R, _D), jnp.float32),
             pltpu.VMEM((2 * _D,), jnp.float32),
             pltpu.VMEM((_ANC_PER_W,), jnp.float32)]
            + [pltpu.SemaphoreType.DMA] * (_K + 1)
        ),
    )(tok_table, x, coef)


def kernel(x, tok_table):
    return _embed(x.astype(jnp.int32), tok_table, _COEF)


# speedup vs baseline: 1.0480x; 1.0480x over previous
import jax
import jax.numpy as jnp
import numpy as np
from jax import lax
from jax.experimental import pallas as pl
from jax.experimental.pallas import tpu as pltpu
from jax.experimental.pallas import tpu_sc as plsc

_VOCAB = 1000000
_D = 128
_SEQ = 8192

_NC = 2
_NS = 16
_L = 16
_NW = _NC * _NS
_B_PER_W = _SEQ // _NW
_K = 4
_R = _B_PER_W // _K
_NCH = _D // _L


def _coef_table():
    col = np.arange(_D)
    w = np.power(10000.0, -2.0 * (col // 2) / _D)
    even = (col % 2) == 0

    steps = np.concatenate([np.cos(w), np.sin(w)])

    anchors = np.empty((_NW, _K, 2, _D))
    for wid in range(_NW):
        for k in range(_K):
            p0 = wid * _B_PER_W + k * _R
            s, c = np.sin(p0 * w), np.cos(p0 * w)
            anchors[wid, k, 0] = np.where(even, s, c)
            anchors[wid, k, 1] = np.where(even, c, -s)
    flat = np.concatenate([steps, anchors.reshape(-1)])
    return flat.astype(np.float32)


_COEF = jnp.asarray(_coef_table())
_ANC_PER_W = _K * 2 * _D


def _body(tok_hbm, idx_hbm, coef_hbm, out_hbm,
          idx_v, rows_v, step_v, anc_v, *sems):
    gsems, wsem = sems[:_K], sems[_K]
    wid = lax.axis_index("s") * _NC + lax.axis_index("c")
    base = wid * _B_PER_W

    pltpu.sync_copy(idx_hbm.at[pl.ds(base, _B_PER_W)], idx_v)
    gathers = [
        pltpu.async_copy(tok_hbm.at[idx_v.at[pl.ds(k * _R, _R)]],
                         rows_v.at[k], gsems[k])
        for k in range(_K)
    ]
    csem = sems[_K + 1]
    c0 = pltpu.async_copy(coef_hbm.at[pl.ds(0, 2 * _D)], step_v, csem)
    c1 = pltpu.async_copy(
        coef_hbm.at[pl.ds(2 * _D + wid * _ANC_PER_W, _ANC_PER_W)], anc_v,
        csem)
    c0.wait()
    c1.wait()

    cw = [step_v[pl.ds(c * _L, _L)] for c in range(_NCH)]
    sw = [step_v[pl.ds(_D + c * _L, _L)] for c in range(_NCH)]

    def add_chunk(k):
        def add_row(r, st):
            v, u = st
            nv, nu = [], []
            for c in range(_NCH):
                plsc.addupdate(rows_v.at[k, r, pl.ds(c * _L, _L)], v[c])
                nv.append(v[c] * cw[c] + u[c] * sw[c])
                nu.append(u[c] * cw[c] - v[c] * sw[c])
            return tuple(nv), tuple(nu)

        v0 = tuple(anc_v[pl.ds(k * 2 * _D + c * _L, _L)]
                   for c in range(_NCH))
        u0 = tuple(anc_v[pl.ds(k * 2 * _D + _D + c * _L, _L)]
                   for c in range(_NCH))
        lax.fori_loop(0, _R, add_row, (v0, u0), unroll=2)

    writes = []
    for k in range(_K):
        gathers[k].wait()
        add_chunk(k)
        writes.append(pltpu.async_copy(
            rows_v.at[k], out_hbm.at[pl.ds(base + k * _R, _R)], wsem))
    for wdma in writes:
        wdma.wait()


def _embed(x, tok_table, coef):
    mesh = plsc.VectorSubcoreMesh(
        core_axis_name="c", subcore_axis_name="s",
        num_cores=_NC, num_subcores=_NS)
    return pl.kernel(
        _body,
        out_type=jax.ShapeDtypeStruct((_SEQ, _D), jnp.float32),
        mesh=mesh,
        scratch_types=(
            [pltpu.VMEM((_B_PER_W,), jnp.int32),
             pltpu.VMEM((_K, _R, _D), jnp.float32),
             pltpu.VMEM((2 * _D,), jnp.float32),
             pltpu.VMEM((_ANC_PER_W,), jnp.float32)]
            + [pltpu.SemaphoreType.DMA] * (_K + 2)
        ),
    )(tok_table, x, coef)


def kernel(x, tok_table):
    return _embed(x.astype(jnp.int32), tok_table, _COEF)
